# baseline (speedup 1.0000x reference)
import functools

import jax
import jax.numpy as jnp
from jax import lax
from jax.experimental import pallas as pl
from jax.experimental.pallas import tpu as pltpu
from jax.experimental.pallas import tpu_sc as plsc

EMB_SIZE = 1000000
EMB_DIM = 64
BATCH = 16384
NEG = 5
NSC = 6
PITCH = 128

_info = plsc.get_sparse_core_info()
NC = _info.num_cores
NS = _info.num_subcores
NW = NC * NS
BPW = BATCH // NW
NPW = BPW * NEG
CHUNK = 64
NCHUNK = BPW // CHUNK
NEG_CHUNK = CHUNK * NEG


def _sc_scores(pos_u, pos_v, neg_flat, u_pad, v_pad):
    mesh = plsc.VectorSubcoreMesh(core_axis_name="c", subcore_axis_name="s")

    @functools.partial(
        pl.kernel,
        mesh=mesh,
        out_type=jax.ShapeDtypeStruct((NW * NSC * BPW,), jnp.float32),
        scratch_types=[
            pltpu.VMEM((BPW,), jnp.int32),
            pltpu.VMEM((BPW,), jnp.int32),
            pltpu.VMEM((NPW,), jnp.int32),
            pltpu.VMEM((CHUNK, PITCH), jnp.float32),
            pltpu.VMEM((CHUNK, PITCH), jnp.float32),
            pltpu.VMEM((NEG_CHUNK, PITCH), jnp.float32),
            pltpu.VMEM((NSC * BPW,), jnp.float32),
            pltpu.SemaphoreType.DMA,
        ],
        compiler_params=pltpu.CompilerParams(
            needs_layout_passes=False, use_tc_tiling_on_sc=True),
    )
    def kern(pos_u_h, pos_v_h, neg_h, u_w, v_w, out_h,
             idx_u, idx_v, idx_n, u_rows, v_rows, n_rows, scores, sem):
        wid = lax.axis_index("s") * NC + lax.axis_index("c")
        base = wid * BPW
        pltpu.sync_copy(pos_u_h.at[pl.ds(base, BPW)], idx_u)
        pltpu.sync_copy(pos_v_h.at[pl.ds(base, BPW)], idx_v)
        pltpu.sync_copy(neg_h.at[pl.ds(base * NEG, NPW)], idx_n)

        lane = lax.iota(jnp.int32, 16)

        def chunk_body(c, carry):
            cpys = [
                pltpu.async_copy(u_w.at[idx_u.at[pl.ds(c * CHUNK, CHUNK)]],
                                 u_rows, sem),
                pltpu.async_copy(v_w.at[idx_v.at[pl.ds(c * CHUNK, CHUNK)]],
                                 v_rows, sem),
            ]
            off = 0
            while off < NEG_CHUNK:
                ln = min(128, NEG_CHUNK - off)
                cpys.append(pltpu.async_copy(
                    v_w.at[idx_n.at[pl.ds(c * NEG_CHUNK + off, ln)]],
                    n_rows.at[pl.ds(off, ln)], sem))
                off += ln
            for cpy in cpys:
                cpy.wait()

            def group_body(g, carry2):
                rb = c * CHUNK + g * 16
                acc = [jnp.zeros((16,), jnp.float32) for _ in range(NSC)]
                for r in range(16):
                    row = g * 16 + r
                    u = [u_rows[row, pl.ds(16 * j, 16)] for j in range(4)]
                    v = [v_rows[row, pl.ds(16 * j, 16)] for j in range(4)]
                    m = lane == r
                    s = u[0] * v[0] + u[1] * v[1] + u[2] * v[2] + u[3] * v[3]
                    acc[0] = jnp.where(m, jnp.sum(s), acc[0])
                    for k in range(NEG):
                        nrow = row * NEG + k
                        n = [n_rows[nrow, pl.ds(16 * j, 16)]
                             for j in range(4)]
                        sk = (u[0] * n[0] + u[1] * n[1]
                              + u[2] * n[2] + u[3] * n[3])
                        acc[1 + k] = jnp.where(m, jnp.sum(sk), acc[1 + k])
                for col in range(NSC):
                    scores[pl.ds(col * BPW + rb, 16)] = acc[col]
                return carry2

            lax.fori_loop(0, CHUNK // 16, group_body, 0)
            return carry

        lax.fori_loop(0, NCHUNK, chunk_body, 0)

        pltpu.sync_copy(scores, out_h.at[pl.ds(wid * NSC * BPW, NSC * BPW)])

    return kern(pos_u, pos_v, neg_flat, u_pad, v_pad)


_PAD_ROWS = 2000
_PAD_BLOCKS = EMB_SIZE // _PAD_ROWS


def _tc_pad(table):
    def body(s_ref, o_ref):
        o_ref[:, : EMB_DIM] = s_ref[...]

    return pl.pallas_call(
        body,
        grid=(_PAD_BLOCKS,),
        out_shape=jax.ShapeDtypeStruct((EMB_SIZE, PITCH), jnp.float32),
        in_specs=[pl.BlockSpec((_PAD_ROWS, EMB_DIM), lambda i: (i, 0))],
        out_specs=pl.BlockSpec((_PAD_ROWS, PITCH), lambda i: (i, 0)),
    )(table)


_TC_ROWS = BATCH * NSC // 128


def _tc_loss(scores):
    flat = scores.reshape(_TC_ROWS, 128)

    def body(s_ref, o_ref):
        x = s_ref[...]
        idx = (lax.broadcasted_iota(jnp.int32, (_TC_ROWS, 128), 0) * 128
               + lax.broadcasted_iota(jnp.int32, (_TC_ROWS, 128), 1))
        col = (idx // BPW) % NSC
        t = jnp.clip(x, -10.0, 10.0)
        t = jnp.where(col == 0, -t, t)
        contrib = jnp.log(1.0 + jnp.exp(t))
        o_ref[0, 0] = jnp.sum(contrib) / BATCH

    return pl.pallas_call(
        body,
        out_shape=jax.ShapeDtypeStruct((1, 1), jnp.float32),
        in_specs=[pl.BlockSpec((_TC_ROWS, 128), lambda: (0, 0))],
        out_specs=pl.BlockSpec(memory_space=pltpu.SMEM),
    )(flat)


def kernel(pos_u, pos_v, neg_v, u_weight, v_weight):
    pos_u = pos_u.astype(jnp.int32)
    pos_v = pos_v.astype(jnp.int32)
    neg_flat = neg_v.reshape(-1).astype(jnp.int32)
    u_pad = _tc_pad(u_weight)
    v_pad = _tc_pad(v_weight)
    scores = _sc_scores(pos_u, pos_v, neg_flat, u_pad, v_pad)
    return _tc_loss(scores)[0, 0]


# speedup vs baseline: 1.6179x; 1.6179x over previous
import functools

import jax
import jax.numpy as jnp
from jax import lax
from jax.experimental import pallas as pl
from jax.experimental.pallas import tpu as pltpu
from jax.experimental.pallas import tpu_sc as plsc

EMB_SIZE = 1000000
EMB_DIM = 64
BATCH = 16384
NEG = 5
NSC = 6
PITCH = 128

_info = plsc.get_sparse_core_info()
NC = _info.num_cores
NS = _info.num_subcores
NW = NC * NS
BPW = BATCH // NW
NPW = BPW * NEG
CHUNK = 64
NCHUNK = BPW // CHUNK
NEG_CHUNK = CHUNK * NEG


def _sc_scores(pos_u, pos_v, neg_flat, u_pad, v_pad):
    mesh = plsc.VectorSubcoreMesh(core_axis_name="c", subcore_axis_name="s")

    @functools.partial(
        pl.kernel,
        mesh=mesh,
        out_type=jax.ShapeDtypeStruct((NW * NSC * BPW,), jnp.float32),
        scratch_types=[
            pltpu.VMEM((BPW,), jnp.int32),
            pltpu.VMEM((BPW,), jnp.int32),
            pltpu.VMEM((NPW,), jnp.int32),
            pltpu.VMEM((CHUNK, PITCH), jnp.float32),
            pltpu.VMEM((CHUNK, PITCH), jnp.float32),
            pltpu.VMEM((NEG_CHUNK, PITCH), jnp.float32),
            pltpu.VMEM((NSC * BPW,), jnp.float32),
            pltpu.SemaphoreType.DMA,
        ],
        compiler_params=pltpu.CompilerParams(
            needs_layout_passes=False, use_tc_tiling_on_sc=True),
    )
    def kern(pos_u_h, pos_v_h, neg_h, u_w, v_w, out_h,
             idx_u, idx_v, idx_n, u_rows, v_rows, n_rows, scores, sem):
        wid = lax.axis_index("s") * NC + lax.axis_index("c")
        base = wid * BPW
        pltpu.sync_copy(pos_u_h.at[pl.ds(base, BPW)], idx_u)
        pltpu.sync_copy(pos_v_h.at[pl.ds(base, BPW)], idx_v)
        pltpu.sync_copy(neg_h.at[pl.ds(base * NEG, NPW)], idx_n)

        lane = lax.iota(jnp.int32, 16)

        def chunk_body(c, carry):
            cpys = [
                pltpu.async_copy(u_w.at[idx_u.at[pl.ds(c * CHUNK, CHUNK)]],
                                 u_rows, sem),
                pltpu.async_copy(v_w.at[idx_v.at[pl.ds(c * CHUNK, CHUNK)]],
                                 v_rows, sem),
            ]
            off = 0
            while off < NEG_CHUNK:
                ln = min(128, NEG_CHUNK - off)
                cpys.append(pltpu.async_copy(
                    v_w.at[idx_n.at[pl.ds(c * NEG_CHUNK + off, ln)]],
                    n_rows.at[pl.ds(off, ln)], sem))
                off += ln
            for cpy in cpys:
                cpy.wait()

            def group_body(g, carry2):
                rb = c * CHUNK + g * 16
                acc = [jnp.zeros((16,), jnp.float32) for _ in range(NSC)]
                for r in range(16):
                    row = g * 16 + r
                    u = [u_rows[row, pl.ds(16 * j, 16)] for j in range(4)]
                    v = [v_rows[row, pl.ds(16 * j, 16)] for j in range(4)]
                    m = lane == r
                    s = u[0] * v[0] + u[1] * v[1] + u[2] * v[2] + u[3] * v[3]
                    acc[0] = jnp.where(m, jnp.sum(s), acc[0])
                    for k in range(NEG):
                        nrow = row * NEG + k
                        n = [n_rows[nrow, pl.ds(16 * j, 16)]
                             for j in range(4)]
                        sk = (u[0] * n[0] + u[1] * n[1]
                              + u[2] * n[2] + u[3] * n[3])
                        acc[1 + k] = jnp.where(m, jnp.sum(sk), acc[1 + k])
                for col in range(NSC):
                    scores[pl.ds(col * BPW + rb, 16)] = acc[col]
                return carry2

            lax.fori_loop(0, CHUNK // 16, group_body, 0)
            return carry

        lax.fori_loop(0, NCHUNK, chunk_body, 0)

        pltpu.sync_copy(scores, out_h.at[pl.ds(wid * NSC * BPW, NSC * BPW)])

    return kern(pos_u, pos_v, neg_flat, u_pad, v_pad)


_TC_ROWS = BATCH * NSC // 128


def _tc_loss(scores):
    flat = scores.reshape(_TC_ROWS, 128)

    def body(s_ref, o_ref):
        x = s_ref[...]
        idx = (lax.broadcasted_iota(jnp.int32, (_TC_ROWS, 128), 0) * 128
               + lax.broadcasted_iota(jnp.int32, (_TC_ROWS, 128), 1))
        col = (idx // BPW) % NSC
        t = jnp.clip(x, -10.0, 10.0)
        t = jnp.where(col == 0, -t, t)
        contrib = jnp.log(1.0 + jnp.exp(t))
        o_ref[0, 0] = jnp.sum(contrib) / BATCH

    return pl.pallas_call(
        body,
        out_shape=jax.ShapeDtypeStruct((1, 1), jnp.float32),
        in_specs=[pl.BlockSpec((_TC_ROWS, 128), lambda: (0, 0))],
        out_specs=pl.BlockSpec(memory_space=pltpu.SMEM),
    )(flat)


def kernel(pos_u, pos_v, neg_v, u_weight, v_weight):
    pos_u = pos_u.astype(jnp.int32)
    pos_v = pos_v.astype(jnp.int32)
    neg_flat = neg_v.reshape(-1).astype(jnp.int32)
    u_pad = jnp.pad(u_weight, ((0, 0), (0, PITCH - EMB_DIM)))
    v_pad = jnp.pad(v_weight, ((0, 0), (0, PITCH - EMB_DIM)))
    scores = _sc_scores(pos_u, pos_v, neg_flat, u_pad, v_pad)
    return _tc_loss(scores)[0, 0]
